# Initial kernel scaffold; baseline (speedup 1.0000x reference)
import jax
import jax.numpy as jnp
from jax.experimental import pallas as pl


def kernel(U, ref):
    raise NotImplementedError("write your pallas kernel here")


# speedup vs baseline: 1.1231x; 1.1231x over previous
import jax
import jax.numpy as jnp
from jax.experimental import pallas as pl

_RB = 1024
_CB = 1024


def _body(a_ref, bt_ref, u_ref, o_ref):
    i = pl.program_id(0)
    n = bt_ref.shape[1]
    c = u_ref.shape[1]
    a = a_ref[:, :]
    sqi = jnp.sum(a * a, axis=1, keepdims=True)
    acc = jnp.zeros((_RB, c), jnp.float32)
    for j in range(n // _CB):
        bt = bt_ref[:, j * _CB:(j + 1) * _CB]
        sqj = jnp.sum(bt * bt, axis=0, keepdims=True)
        mm = jax.lax.dot_general(a, bt, (((1,), (0,)), ((), ())),
                                 preferred_element_type=jnp.float32)
        d2 = (sqi + sqj) - 2.0 * mm
        w = jnp.exp(-0.5 * jnp.maximum(d2, 0.0))
        acc = acc + jax.lax.dot_general(
            w, u_ref[j * _CB:(j + 1) * _CB, :],
            (((1,), (0,)), ((), ())),
            preferred_element_type=jnp.float32)
    o_ref[:, :] = acc - u_ref[pl.ds(i * _RB, _RB), :]


def kernel(U, ref):
    n, c = U.shape
    refp = jnp.pad(ref, ((0, 0), (0, 8 - ref.shape[1])))
    refT = refp.T

    out = pl.pallas_call(
        _body,
        grid=(n // _RB,),
        in_specs=[
            pl.BlockSpec((_RB, 8), lambda i: (i, 0)),
            pl.BlockSpec((8, n), lambda i: (0, 0)),
            pl.BlockSpec((n, c), lambda i: (0, 0)),
        ],
        out_specs=pl.BlockSpec((_RB, c), lambda i: (i, 0)),
        out_shape=jax.ShapeDtypeStruct((n, c), jnp.float32),
    )(refp, refT, U)
    return out


# speedup vs baseline: 1.1337x; 1.0095x over previous
import jax
import jax.numpy as jnp
from jax.experimental import pallas as pl

_RB = 1024
_CB = 1024
_LOG2E = 1.4426950408889634


def _body(a_ref, bt_ref, u_ref, o_ref):
    i = pl.program_id(0)
    n = bt_ref.shape[1]
    c = u_ref.shape[1]
    a = a_ref[:, :]
    ci = jnp.sum(a * a, axis=1, keepdims=True) * (0.5 * _LOG2E)
    acc = jnp.zeros((_RB, c), jnp.float32)
    for j in range(n // _CB):
        bt = bt_ref[:, j * _CB:(j + 1) * _CB]
        cj = jnp.sum(bt * bt, axis=0, keepdims=True) * (0.5 * _LOG2E)
        mm = jax.lax.dot_general(a.astype(jnp.bfloat16), bt.astype(jnp.bfloat16),
                                 (((1,), (0,)), ((), ())),
                                 preferred_element_type=jnp.float32)
        w = jnp.exp2(jnp.minimum(mm * _LOG2E - (ci + cj), 0.0))
        acc = acc + jax.lax.dot_general(
            w, u_ref[j * _CB:(j + 1) * _CB, :],
            (((1,), (0,)), ((), ())),
            preferred_element_type=jnp.float32)
    o_ref[:, :] = acc - u_ref[pl.ds(i * _RB, _RB), :]


def kernel(U, ref):
    n, c = U.shape
    refp = jnp.pad(ref, ((0, 0), (0, 8 - ref.shape[1])))
    refT = refp.T

    out = pl.pallas_call(
        _body,
        grid=(n // _RB,),
        in_specs=[
            pl.BlockSpec((_RB, 8), lambda i: (i, 0)),
            pl.BlockSpec((8, n), lambda i: (0, 0)),
            pl.BlockSpec((n, c), lambda i: (0, 0)),
        ],
        out_specs=pl.BlockSpec((_RB, c), lambda i: (i, 0)),
        out_shape=jax.ShapeDtypeStruct((n, c), jnp.float32),
    )(refp, refT, U)
    return out


# speedup vs baseline: 1.3441x; 1.1855x over previous
import jax
import jax.numpy as jnp
from jax.experimental import pallas as pl

_RB = 1024
_CB = 512
_NT = 8
_LOG2E = 1.4426950408889634


def _body(a_ref, bt_ref, u_ref, o_ref):
    s = pl.program_id(0)

    @pl.when(s == 0)
    def _init():
        o_ref[...] = jnp.zeros_like(o_ref)

    for t in range(_NT + 1):
        first = t < _NT - s
        i_t = jnp.where(first, s, _NT - 1 - s)
        j_t = jnp.where(first, s + t, t - 1)
        row = i_t * _RB
        a = a_ref[pl.ds(row, _RB), :]
        a16 = a.astype(jnp.bfloat16)
        ci = jnp.sum(a * a, axis=1, keepdims=True) * (0.5 * _LOG2E)
        mirror = jnp.where(j_t > i_t, 1.0, 0.0)
        for k in range(_RB // _CB):
            col = j_t * _RB + k * _CB
            bt = bt_ref[:, pl.ds(col, _CB)]
            cj = jnp.sum(bt * bt, axis=0, keepdims=True) * (0.5 * _LOG2E)
            mm = jax.lax.dot_general(
                a16, bt.astype(jnp.bfloat16),
                (((1,), (0,)), ((), ())),
                preferred_element_type=jnp.float32)
            w = jnp.exp2(jnp.minimum(mm * _LOG2E - (ci + cj), 0.0))
            o_ref[pl.ds(row, _RB), :] += jax.lax.dot_general(
                w, u_ref[pl.ds(col, _CB), :],
                (((1,), (0,)), ((), ())),
                preferred_element_type=jnp.float32)
            o_ref[pl.ds(col, _CB), :] += mirror * jax.lax.dot_general(
                w, u_ref[pl.ds(row, _RB), :],
                (((0,), (0,)), ((), ())),
                preferred_element_type=jnp.float32)

    @pl.when(s == _NT // 2 - 1)
    def _finish():
        o_ref[...] -= u_ref[...]


def kernel(U, ref):
    n, c = U.shape
    refp = jnp.pad(ref, ((0, 0), (0, 8 - ref.shape[1])))
    refT = refp.T

    out = pl.pallas_call(
        _body,
        grid=(_NT // 2,),
        in_specs=[
            pl.BlockSpec((n, 8), lambda i: (0, 0)),
            pl.BlockSpec((8, n), lambda i: (0, 0)),
            pl.BlockSpec((n, c), lambda i: (0, 0)),
        ],
        out_specs=pl.BlockSpec((n, c), lambda i: (0, 0)),
        out_shape=jax.ShapeDtypeStruct((n, c), jnp.float32),
    )(refp, refT, U)
    return out


# speedup vs baseline: 1.3772x; 1.0246x over previous
import jax
import jax.numpy as jnp
from jax.experimental import pallas as pl

_RB = 1024
_CB = 512
_NT = 8
_LOG2E = 1.4426950408889634


def _body(a_ref, bt_ref, u_ref, o_ref):
    s = pl.program_id(0)

    @pl.when(s == 0)
    def _init():
        o_ref[...] = jnp.zeros_like(o_ref)

    for t in range(_NT + 1):
        first = t < _NT - s
        i_t = jnp.where(first, s, _NT - 1 - s)
        j_t = jnp.where(first, s + t, t - 1)
        row = i_t * _RB
        a = a_ref[pl.ds(row, _RB), :]
        a16 = a.astype(jnp.bfloat16)
        ci = jnp.sum(a * a, axis=1, keepdims=True) * (0.5 * _LOG2E)
        mirror = jnp.where(j_t > i_t, 1.0, 0.0)
        for k in range(_RB // _CB):
            col = j_t * _RB + k * _CB
            bt = bt_ref[:, pl.ds(col, _CB)]
            cj = jnp.sum(bt * bt, axis=0, keepdims=True) * (0.5 * _LOG2E)
            mm = jax.lax.dot_general(
                a16, bt.astype(jnp.bfloat16),
                (((1,), (0,)), ((), ())),
                preferred_element_type=jnp.float32)
            w = jnp.exp2(jnp.minimum(mm * _LOG2E - (ci + cj), 0.0)
                         ).astype(jnp.bfloat16)
            o_ref[pl.ds(row, _RB), :] += jax.lax.dot_general(
                w, u_ref[pl.ds(col, _CB), :],
                (((1,), (0,)), ((), ())),
                preferred_element_type=jnp.float32)
            o_ref[pl.ds(col, _CB), :] += mirror * jax.lax.dot_general(
                w, u_ref[pl.ds(row, _RB), :],
                (((0,), (0,)), ((), ())),
                preferred_element_type=jnp.float32)

    @pl.when(s == _NT // 2 - 1)
    def _finish():
        o_ref[...] -= u_ref[...]


def kernel(U, ref):
    n, c = U.shape
    refp = jnp.pad(ref, ((0, 0), (0, 8 - ref.shape[1])))
    refT = refp.T

    out = pl.pallas_call(
        _body,
        grid=(_NT // 2,),
        in_specs=[
            pl.BlockSpec((n, 8), lambda i: (0, 0)),
            pl.BlockSpec((8, n), lambda i: (0, 0)),
            pl.BlockSpec((n, c), lambda i: (0, 0)),
        ],
        out_specs=pl.BlockSpec((n, c), lambda i: (0, 0)),
        out_shape=jax.ShapeDtypeStruct((n, c), jnp.float32),
    )(refp, refT, U)
    return out
